# Initial kernel scaffold; baseline (speedup 1.0000x reference)
import jax
import jax.numpy as jnp
from jax.experimental import pallas as pl


def kernel(params, h1_index, edge_index1, h2_index, edge_index2):
    raise NotImplementedError("write your pallas kernel here")


# speedup vs baseline: 5.9728x; 5.9728x over previous
import functools

import jax
import jax.numpy as jnp
from jax import lax
from jax.experimental import pallas as pl
from jax.experimental.pallas import tpu as pltpu
from jax.experimental.pallas import tpu_sc as plsc

F32 = jnp.float32
N = 10000
NP = 10240
E = 160000
EP = 163840
HID = 128
NW = 32


def _sc_embed_deg(embed, idx_pad, dst_all, ones16, zeros16, iota_all):
    mesh = plsc.VectorSubcoreMesh(core_axis_name="c", subcore_axis_name="s")

    @functools.partial(
        pl.kernel,
        mesh=mesh,
        out_type=[jax.ShapeDtypeStruct((2 * NP, HID), F32),
                  jax.ShapeDtypeStruct((2 * NP, 16), F32),
                  jax.ShapeDtypeStruct((2 * NP, 16), F32)],
        scratch_types=[pltpu.VMEM((128,), jnp.int32),
                       pltpu.VMEM((128, HID), F32),
                       pltpu.VMEM((128,), jnp.int32),
                       pltpu.VMEM((128, 16), F32),
                       pltpu.VMEM((128, 16), F32),
                       pltpu.VMEM_SHARED((2 * NP, 16), F32),
                       pltpu.SemaphoreType.DMA],
    )
    def k(embed_h, idx_h, dst_h, ones_h, zeros_h, iota_h,
          emb_o, degp0_o, degp1_o,
          idx_v, rows_v, dst_v, ones_v, zb_v, acc_sh, sem):
        cid = lax.axis_index("c")
        sid = lax.axis_index("s")
        w = sid * 2 + cid

        def emb_body(j, _):
            base = w * 640 + j * 128
            pltpu.sync_copy(idx_h.at[pl.ds(base, 128)], idx_v)
            pltpu.async_copy(embed_h.at[idx_v], rows_v, sem).wait()
            pltpu.sync_copy(rows_v, emb_o.at[pl.ds(base, 128)])
            return 0

        lax.fori_loop(0, 5, emb_body, 0, unroll=False)

        pltpu.sync_copy(zeros_h.at[pl.ds(0, 128)], zb_v)
        pltpu.sync_copy(ones_h, ones_v)

        def zero_body(j, _):
            off = sid * 1280 + j * 128
            pltpu.sync_copy(iota_h.at[pl.ds(off, 128)], idx_v)
            pltpu.sync_copy(zb_v, acc_sh.at[idx_v])
            return 0

        lax.fori_loop(0, 10, zero_body, 0, unroll=False)
        plsc.subcore_barrier()

        def deg_body(j, _):
            base = w * 10240 + j * 128
            pltpu.sync_copy(dst_h.at[pl.ds(base, 128)], dst_v)
            pltpu.sync_copy(ones_v, acc_sh.at[dst_v], add=True)
            return 0

        lax.fori_loop(0, 80, deg_body, 0, unroll=False)
        plsc.subcore_barrier()

        def dump_body(j, _):
            off = sid * 1280 + j * 128
            pltpu.sync_copy(iota_h.at[pl.ds(off, 128)], idx_v)
            pltpu.async_copy(acc_sh.at[idx_v], zb_v, sem).wait()

            @pl.when(cid == 0)
            def _d0():
                pltpu.sync_copy(zb_v, degp0_o.at[pl.ds(off, 128)])

            @pl.when(cid == 1)
            def _d1():
                pltpu.sync_copy(zb_v, degp1_o.at[pl.ds(off, 128)])

            return 0

        lax.fori_loop(0, 10, dump_body, 0, unroll=False)

    emb, d0, d1 = k(embed, idx_pad, dst_all, ones16, zeros16, iota_all)
    return emb, jnp.stack([d0, d1])


def _sc_scatter(y, src_pad, dst_pad, zeros128, iota_np):
    mesh = plsc.VectorSubcoreMesh(core_axis_name="c", subcore_axis_name="s")

    @functools.partial(
        pl.kernel,
        mesh=mesh,
        out_type=[jax.ShapeDtypeStruct((NP, HID), F32),
                  jax.ShapeDtypeStruct((NP, HID), F32)],
        scratch_types=[pltpu.VMEM((128,), jnp.int32),
                       pltpu.VMEM((128,), jnp.int32),
                       pltpu.VMEM((128, HID), F32),
                       pltpu.VMEM((128, HID), F32),
                       pltpu.VMEM_SHARED((NP, HID), F32),
                       pltpu.SemaphoreType.DMA],
    )
    def k(y_h, src_h, dst_h, zeros_h, iota_h, out0_o, out1_o,
          src_v, dst_v, rows_v, zb_v, acc_sh, sem):
        cid = lax.axis_index("c")
        sid = lax.axis_index("s")
        w = sid * 2 + cid

        pltpu.sync_copy(zeros_h.at[pl.ds(0, 128)], zb_v)

        def zero_body(j, _):
            off = sid * 640 + j * 128
            pltpu.sync_copy(iota_h.at[pl.ds(off, 128)], src_v)
            pltpu.sync_copy(zb_v, acc_sh.at[src_v])
            return 0

        lax.fori_loop(0, 5, zero_body, 0, unroll=False)
        plsc.subcore_barrier()

        def body(j, _):
            base = w * 5120 + j * 128
            pltpu.sync_copy(src_h.at[pl.ds(base, 128)], src_v)
            pltpu.sync_copy(dst_h.at[pl.ds(base, 128)], dst_v)
            pltpu.async_copy(y_h.at[src_v], rows_v, sem).wait()
            pltpu.sync_copy(rows_v, acc_sh.at[dst_v], add=True)
            return 0

        lax.fori_loop(0, 40, body, 0, unroll=False)
        plsc.subcore_barrier()

        def dump_body(j, _):
            off = sid * 640 + j * 128
            pltpu.sync_copy(iota_h.at[pl.ds(off, 128)], src_v)
            pltpu.async_copy(acc_sh.at[src_v], zb_v, sem).wait()

            @pl.when(cid == 0)
            def _d0():
                pltpu.sync_copy(zb_v, out0_o.at[pl.ds(off, 128)])

            @pl.when(cid == 1)
            def _d1():
                pltpu.sync_copy(zb_v, out1_o.at[pl.ds(off, 128)])

            return 0

        lax.fori_loop(0, 5, dump_body, 0, unroll=False)

    s0, s1 = k(y, src_pad, dst_pad, zeros128, iota_np)
    return jnp.stack([s0, s1])


_BR = 256


def _tc_dinv_mm1(emb, degp, W1):
    def body(e_ref, d_ref, w_ref, y_ref, dv_ref):
        deg = d_ref[0, :, 0:1] + d_ref[1, :, 0:1] + 1.0
        dinv = lax.rsqrt(deg)
        xw = jnp.dot(e_ref[0], w_ref[...], preferred_element_type=F32)
        y_ref[0] = xw * dinv
        dv_ref[0] = jnp.broadcast_to(dinv, (_BR, HID))

    nb = NP // _BR
    return pl.pallas_call(
        body,
        grid=(2, nb),
        in_specs=[pl.BlockSpec((1, _BR, HID), lambda g, r: (g, r, 0)),
                  pl.BlockSpec((2, _BR, 16), lambda g, r: (0, g * nb + r, 0)),
                  pl.BlockSpec((HID, HID), lambda g, r: (0, 0))],
        out_specs=[pl.BlockSpec((1, _BR, HID), lambda g, r: (g, r, 0))] * 2,
        out_shape=[jax.ShapeDtypeStruct((2, NP, HID), F32)] * 2,
    )(emb, degp, W1)


def _tc_gcn_out_mm2(Sp, y1, dinvb, W2, b1):
    def body(s_ref, y_ref, d_ref, w_ref, b_ref, o_ref, y2_ref):
        S = s_ref[0, 0] + s_ref[1, 0]
        o = jnp.maximum(d_ref[0] * (S + y_ref[0]) + b_ref[...], 0.0)
        o_ref[0] = o
        y2_ref[0] = jnp.dot(o, w_ref[...], preferred_element_type=F32) * d_ref[0]

    return pl.pallas_call(
        body,
        grid=(2, NP // _BR),
        in_specs=[pl.BlockSpec((2, 1, _BR, HID), lambda g, r: (0, g, r, 0)),
                  pl.BlockSpec((1, _BR, HID), lambda g, r: (g, r, 0)),
                  pl.BlockSpec((1, _BR, HID), lambda g, r: (g, r, 0)),
                  pl.BlockSpec((HID, HID), lambda g, r: (0, 0)),
                  pl.BlockSpec((1, HID), lambda g, r: (0, 0))],
        out_specs=[pl.BlockSpec((1, _BR, HID), lambda g, r: (g, r, 0))] * 2,
        out_shape=[jax.ShapeDtypeStruct((2, NP, HID), F32)] * 2,
    )(Sp, y1, dinvb, W2, b1)


def _tc_gcn_out(Sp, y2, dinvb, b2):
    def body(s_ref, y_ref, d_ref, b_ref, o_ref):
        S = s_ref[0, 0] + s_ref[1, 0]
        o_ref[0] = jnp.maximum(d_ref[0] * (S + y_ref[0]) + b_ref[...], 0.0)

    return pl.pallas_call(
        body,
        grid=(2, NP // _BR),
        in_specs=[pl.BlockSpec((2, 1, _BR, HID), lambda g, r: (0, g, r, 0)),
                  pl.BlockSpec((1, _BR, HID), lambda g, r: (g, r, 0)),
                  pl.BlockSpec((1, _BR, HID), lambda g, r: (g, r, 0)),
                  pl.BlockSpec((1, HID), lambda g, r: (0, 0))],
        out_specs=pl.BlockSpec((1, _BR, HID), lambda g, r: (g, r, 0)),
        out_shape=jax.ShapeDtypeStruct((2, NP, HID), F32),
    )(Sp, y2, dinvb, b2)


_TB = 400
_NTB = N // _TB
_G4 = 512


def _tc_proj1(emb, out1, out2, Wf, Wb, bf, bb):
    def body(e_ref, o1_ref, o2_ref, wf_ref, wb_ref, bf_ref, bb_ref,
             xpf_ref, xpb_ref):
        xs = (e_ref[0], o1_ref[0], o2_ref[0])
        for m in range(3):
            f = jnp.dot(xs[m], wf_ref[m], preferred_element_type=F32) + bf_ref[m]
            b = jnp.dot(xs[m], wb_ref[m], preferred_element_type=F32) + bb_ref[m]
            xpf_ref[m, 0] = f
            xpb_ref[m, 0] = b

    return pl.pallas_call(
        body,
        grid=(2, _NTB),
        in_specs=[pl.BlockSpec((1, _TB, HID), lambda g, r: (g, r, 0)),
                  pl.BlockSpec((1, _TB, HID), lambda g, r: (g, r, 0)),
                  pl.BlockSpec((1, _TB, HID), lambda g, r: (g, r, 0)),
                  pl.BlockSpec((3, HID, _G4), lambda g, r: (0, 0, 0)),
                  pl.BlockSpec((3, HID, _G4), lambda g, r: (0, 0, 0)),
                  pl.BlockSpec((3, 1, _G4), lambda g, r: (0, 0, 0)),
                  pl.BlockSpec((3, 1, _G4), lambda g, r: (0, 0, 0))],
        out_specs=[pl.BlockSpec((3, 1, _TB, _G4), lambda g, r: (0, g, r, 0)),
                   pl.BlockSpec((3, 1, _TB, _G4),
                                lambda g, r: (0, g, _NTB - 1 - r, 0))],
        out_shape=[jax.ShapeDtypeStruct((3, 2, N, _G4), F32)] * 2,
    )(emb, out1, out2, Wf, Wb, bf, bb)


def _lstm_cell(gates, c):
    i = jax.nn.sigmoid(gates[:, 0:128])
    f = jax.nn.sigmoid(gates[:, 128:256])
    g = jnp.tanh(gates[:, 256:384])
    o = jax.nn.sigmoid(gates[:, 384:512])
    c2 = f * c + i * g
    return o * jnp.tanh(c2), c2


def _tc_lstm1(xpf, xpb, Whf, Whb):
    def body(xpf_ref, xpb_ref, wf_ref, wb_ref, of_ref, ob_ref, *scr):
        @pl.when(pl.program_id(0) == 0)
        def _init():
            for s in scr:
                s[...] = jnp.zeros((2, HID), F32)

        def step(rr, _):
            for m in range(3):
                hf, cf = scr[m], scr[3 + m]
                hb, cb = scr[6 + m], scr[9 + m]
                gf = xpf_ref[m, :, rr] + jnp.dot(hf[...], wf_ref[m],
                                                 preferred_element_type=F32)
                h2, c2 = _lstm_cell(gf, cf[...])
                hf[...] = h2
                cf[...] = c2
                of_ref[m, :, rr] = h2
                gb = (xpb_ref[m, :, _TB - 1 - rr]
                      + jnp.dot(hb[...], wb_ref[m], preferred_element_type=F32))
                h2b, c2b = _lstm_cell(gb, cb[...])
                hb[...] = h2b
                cb[...] = c2b
                ob_ref[m, :, _TB - 1 - rr] = h2b
            return 0

        lax.fori_loop(0, _TB, step, 0, unroll=False)

    return pl.pallas_call(
        body,
        grid=(_NTB,),
        in_specs=[pl.BlockSpec((3, 2, _TB, _G4), lambda r: (0, 0, r, 0)),
                  pl.BlockSpec((3, 2, _TB, _G4), lambda r: (0, 0, r, 0)),
                  pl.BlockSpec((3, HID, _G4), lambda r: (0, 0, 0)),
                  pl.BlockSpec((3, HID, _G4), lambda r: (0, 0, 0))],
        out_specs=[pl.BlockSpec((3, 2, _TB, HID), lambda r: (0, 0, r, 0)),
                   pl.BlockSpec((3, 2, _TB, HID),
                                lambda r: (0, 0, _NTB - 1 - r, 0))],
        out_shape=[jax.ShapeDtypeStruct((3, 2, N, HID), F32)] * 2,
        scratch_shapes=[pltpu.VMEM((2, HID), F32)] * 12,
    )(xpf, xpb, Whf, Whb)


def _tc_proj2(of, ob, W2f, b2f):
    def body(f_ref, b_ref, w_ref, bias_ref, o_ref):
        for m in range(3):
            xc = jnp.concatenate([f_ref[m, 0], b_ref[m, 0]], axis=1)
            o_ref[m, 0] = (jnp.dot(xc, w_ref[m], preferred_element_type=F32)
                           + bias_ref[m])

    return pl.pallas_call(
        body,
        grid=(2, _NTB),
        in_specs=[pl.BlockSpec((3, 1, _TB, HID), lambda g, r: (0, g, r, 0)),
                  pl.BlockSpec((3, 1, _TB, HID), lambda g, r: (0, g, r, 0)),
                  pl.BlockSpec((3, 2 * HID, _G4), lambda g, r: (0, 0, 0)),
                  pl.BlockSpec((3, 1, _G4), lambda g, r: (0, 0, 0))],
        out_specs=pl.BlockSpec((3, 1, _TB, _G4), lambda g, r: (0, g, r, 0)),
        out_shape=jax.ShapeDtypeStruct((3, 2, N, _G4), F32),
    )(of, ob, W2f, b2f)


def _tc_lstm2(xp2, Wh2):
    def body(xp_ref, w_ref, h_ref, *scr):
        @pl.when(pl.program_id(0) == 0)
        def _init():
            for s in scr:
                s[...] = jnp.zeros((2, HID), F32)

        def step(rr, _):
            for m in range(3):
                h, c = scr[m], scr[3 + m]
                g = xp_ref[m, :, rr] + jnp.dot(h[...], w_ref[m],
                                               preferred_element_type=F32)
                h2, c2 = _lstm_cell(g, c[...])
                h[...] = h2
                c[...] = c2
            return 0

        lax.fori_loop(0, _TB, step, 0, unroll=False)

        @pl.when(pl.program_id(0) == _NTB - 1)
        def _emit():
            for m in range(3):
                h_ref[m] = scr[m][...]

    return pl.pallas_call(
        body,
        grid=(_NTB,),
        in_specs=[pl.BlockSpec((3, 2, _TB, _G4), lambda r: (0, 0, r, 0)),
                  pl.BlockSpec((3, HID, _G4), lambda r: (0, 0, 0))],
        out_specs=pl.BlockSpec((3, 2, HID), lambda r: (0, 0, 0)),
        out_shape=jax.ShapeDtypeStruct((3, 2, HID), F32),
        scratch_shapes=[pltpu.VMEM((2, HID), F32)] * 6,
    )(xp2, Wh2)


def _tc_final(of, ob, h2f, Wb2, bb2, fcm, fcmb,
              fuWf1, fuWb1, fuWhf1, fuWhb1, fubf1, fubb1,
              fuWf2, fuWb2, fuWhf2, fubf2, fubb2,
              fuFc, fuFcb, Wfin, bfin):
    def cell(x, h, c, Wih, Whh, b):
        g = (jnp.dot(x, Wih, preferred_element_type=F32)
             + jnp.dot(h, Whh, preferred_element_type=F32) + b)
        return _lstm_cell(g, c)

    def body(of_ref, ob_ref, h2_ref, wb2_ref, bb2_ref, fcm_ref, fcmb_ref,
             wf1_ref, wbk1_ref, whf1_ref, whb1_ref, bf1_ref, bb1_ref,
             wf2_ref, wbk2_ref, whf2_ref, bf2_ref, bbk2_ref,
             fc_ref, fcb_ref, wfin_ref, bfin_ref, out_ref):
        z1 = jnp.zeros((1, HID), F32)
        seqs = []
        for m in range(3):
            x1l = jnp.concatenate([of_ref[m, :, 7], ob_ref[m, :, 7]], axis=1)
            gb = jnp.dot(x1l, wb2_ref[m], preferred_element_type=F32) + bb2_ref[m]
            hb2, _ = _lstm_cell(gb, jnp.zeros((2, HID), F32))
            xm = jnp.concatenate([h2_ref[m], hb2], axis=1)
            om = jnp.dot(xm, fcm_ref[m], preferred_element_type=F32) + fcmb_ref[m]
            seqs.append(jnp.concatenate([om[0:1], om[1:2]], axis=1))
        seq = seqs

        hf, cf = z1, z1
        fs = []
        for t in range(3):
            hf, cf = cell(seq[t], hf, cf, wf1_ref[...], whf1_ref[...], bf1_ref[...])
            fs.append(hf)
        hb, cb = z1, z1
        bs = [None] * 3
        for t in (2, 1, 0):
            hb, cb = cell(seq[t], hb, cb, wbk1_ref[...], whb1_ref[...], bb1_ref[...])
            bs[t] = hb
        x1 = [jnp.concatenate([fs[t], bs[t]], axis=1) for t in range(3)]

        hf2, cf2 = z1, z1
        for t in range(3):
            hf2, cf2 = cell(x1[t], hf2, cf2, wf2_ref[...], whf2_ref[...], bf2_ref[...])
        g2 = jnp.dot(x1[2], wbk2_ref[...], preferred_element_type=F32) + bbk2_ref[...]
        hb2f, _ = _lstm_cell(g2, z1)

        xlast = jnp.concatenate([hf2, hb2f], axis=1)
        fo = jnp.dot(xlast, fc_ref[...], preferred_element_type=F32) + fcb_ref[...]
        z = jnp.dot(fo, wfin_ref[...], preferred_element_type=F32) + bfin_ref[...]
        z2 = z[:, 0:2]
        mx = jnp.max(z2, axis=-1, keepdims=True)
        e = jnp.exp(z2 - mx)
        out_ref[...] = e / jnp.sum(e, axis=-1, keepdims=True)

    lastblk = N // 8 - 1
    spec_last = pl.BlockSpec((3, 2, 8, HID), lambda i: (0, 0, lastblk, 0))
    whole = lambda a: pl.BlockSpec(a.shape, lambda i: (0,) * a.ndim)
    args = (h2f, Wb2, bb2, fcm, fcmb, fuWf1, fuWb1, fuWhf1, fuWhb1, fubf1,
            fubb1, fuWf2, fuWb2, fuWhf2, fubf2, fubb2, fuFc, fuFcb, Wfin, bfin)
    return pl.pallas_call(
        body,
        grid=(1,),
        in_specs=[spec_last, spec_last] + [whole(a) for a in args],
        out_specs=pl.BlockSpec((1, 2), lambda i: (0, 0)),
        out_shape=jax.ShapeDtypeStruct((1, 2), F32),
    )(of, ob, *args)


def _asm_ih(Wih):
    Wt = Wih.T
    return jnp.concatenate(
        [jnp.pad(Wt[:, 100 * q:100 * (q + 1)], ((0, 0), (0, 28)))
         for q in range(4)], axis=1)


def _asm_ih2(Wih):
    Wt = Wih.T
    W = jnp.concatenate([jnp.pad(Wt[0:100], ((0, 28), (0, 0))),
                         jnp.pad(Wt[100:200], ((0, 28), (0, 0)))], axis=0)
    return jnp.concatenate(
        [jnp.pad(W[:, 100 * q:100 * (q + 1)], ((0, 0), (0, 28)))
         for q in range(4)], axis=1)


def _asm_hh(Whh):
    Wt = jnp.pad(Whh.T, ((0, 28), (0, 0)))
    return jnp.concatenate(
        [jnp.pad(Wt[:, 100 * q:100 * (q + 1)], ((0, 0), (0, 28)))
         for q in range(4)], axis=1)


def _asm_b(bih, bhh):
    b = bih + bhh
    return jnp.concatenate(
        [jnp.pad(b[100 * q:100 * (q + 1)], (0, 28)) for q in range(4)])[None]


def _asm_fc(fcW):
    Wt = fcW.T
    return jnp.concatenate([jnp.pad(Wt[0:100], ((0, 28), (0, 0))),
                            jnp.pad(Wt[100:200], ((0, 28), (0, 0)))], axis=0)


def kernel(params, h1_index, edge_index1, h2_index, edge_index2):
    i32 = jnp.int32

    idx_pad = jnp.concatenate(
        [h1_index, h2_index, jnp.zeros((2 * (NP - N),), i32)])
    pad_e = EP - E
    junk = NP - 1

    def pad_edges(ei):
        src = jnp.concatenate([ei[0], jnp.zeros((pad_e,), i32)])
        dst = jnp.concatenate([ei[1], jnp.full((pad_e,), junk, i32)])
        return src, dst

    src1, dst1 = pad_edges(edge_index1)
    src2, dst2 = pad_edges(edge_index2)
    dst_all = jnp.concatenate([dst1, dst2 + NP])

    ones16 = jnp.ones((128, 16), F32)
    zeros16 = jnp.zeros((2 * NP, 16), F32)
    zeros128 = jnp.zeros((NP, HID), F32)
    iota_all = jnp.arange(2 * NP, dtype=i32)

    emb_flat, degp = _sc_embed_deg(params["embed"], idx_pad, dst_all,
                                   ones16, zeros16, iota_all)
    emb = emb_flat.reshape(2, NP, HID)

    y1, dinvb = _tc_dinv_mm1(emb, degp, params["gcn1_W"])
    iota_np = iota_all[:NP]
    S1 = jnp.stack([_sc_scatter(y1[0], src1, dst1, zeros128, iota_np),
                    _sc_scatter(y1[1], src2, dst2, zeros128, iota_np)], axis=1)
    out1, y2 = _tc_gcn_out_mm2(S1, y1, dinvb, params["gcn2_W"],
                               params["gcn1_b"][None])

    S2 = jnp.stack([_sc_scatter(y2[0], src1, dst1, zeros128, iota_np),
                    _sc_scatter(y2[1], src2, dst2, zeros128, iota_np)], axis=1)
    out2 = _tc_gcn_out(S2, y2, dinvb, params["gcn2_b"][None])

    models = [params["lstm1"], params["lstm2"], params["lstm3"]]
    L1 = [p["layers"][0] for p in models]
    L2 = [p["layers"][1] for p in models]
    Wf1 = jnp.stack([_asm_ih(l["f_Wih"]) for l in L1])
    Wb1 = jnp.stack([_asm_ih(l["b_Wih"]) for l in L1])
    bf1 = jnp.stack([_asm_b(l["f_bih"], l["f_bhh"]) for l in L1])
    bb1 = jnp.stack([_asm_b(l["b_bih"], l["b_bhh"]) for l in L1])
    Whf1 = jnp.stack([_asm_hh(l["f_Whh"]) for l in L1])
    Whb1 = jnp.stack([_asm_hh(l["b_Whh"]) for l in L1])
    W2f = jnp.stack([_asm_ih2(l["f_Wih"]) for l in L2])
    b2f = jnp.stack([_asm_b(l["f_bih"], l["f_bhh"]) for l in L2])
    Wh2f = jnp.stack([_asm_hh(l["f_Whh"]) for l in L2])
    Wb2 = jnp.stack([_asm_ih2(l["b_Wih"]) for l in L2])
    bb2 = jnp.stack([_asm_b(l["b_bih"], l["b_bhh"]) for l in L2])
    fcm = jnp.stack([_asm_fc(p["fc_W"]) for p in models])
    fcmb = jnp.stack([p["fc_b"][None] for p in models])

    fu = params["fusion"]
    fL1, fL2 = fu["layers"]
    fuWf1 = _asm_ih(fL1["f_Wih"])
    fuWb1 = _asm_ih(fL1["b_Wih"])
    fuWhf1 = _asm_hh(fL1["f_Whh"])
    fuWhb1 = _asm_hh(fL1["b_Whh"])
    fubf1 = _asm_b(fL1["f_bih"], fL1["f_bhh"])
    fubb1 = _asm_b(fL1["b_bih"], fL1["b_bhh"])
    fuWf2 = _asm_ih2(fL2["f_Wih"])
    fuWb2 = _asm_ih2(fL2["b_Wih"])
    fuWhf2 = _asm_hh(fL2["f_Whh"])
    fubf2 = _asm_b(fL2["f_bih"], fL2["f_bhh"])
    fubb2 = _asm_b(fL2["b_bih"], fL2["b_bhh"])
    fuFc = _asm_fc(fu["fc_W"])
    fuFcb = fu["fc_b"][None]
    Wfin = jnp.pad(params["fc_W"].T, ((0, 0), (0, HID - 2)))
    bfin = jnp.pad(params["fc_b"], (0, HID - 2))[None]

    xpf, xpb = _tc_proj1(emb, out1, out2, Wf1, Wb1, bf1, bb1)
    of, ob = _tc_lstm1(xpf, xpb, Whf1, Whb1)
    xp2 = _tc_proj2(of, ob, W2f, b2f)
    h2f = _tc_lstm2(xp2, Wh2f)

    return _tc_final(of, ob, h2f, Wb2, bb2, fcm, fcmb,
                     fuWf1, fuWb1, fuWhf1, fuWhb1, fubf1, fubb1,
                     fuWf2, fuWb2, fuWhf2, fubf2, fubb2,
                     fuFc, fuFcb, Wfin, bfin)
